# Initial kernel scaffold; baseline (speedup 1.0000x reference)
import jax
import jax.numpy as jnp
from jax.experimental import pallas as pl


def kernel(detection_boxes, detection_labels, regression, classification, anchors):
    raise NotImplementedError("write your pallas kernel here")


# speedup vs baseline: 1.3882x; 1.3882x over previous
import jax
import jax.numpy as jnp
from jax import lax
from jax.experimental import pallas as pl
from jax.experimental.pallas import tpu as pltpu

_BLKA = 2000


def _fused_body(boxes_ref, labs_ref, anch_ref, reg_ref, cls_ref,
                cls_out, det_out, acc_ref):
    j = pl.program_id(0)
    b = pl.program_id(1)
    nbatch = pl.num_programs(0)
    nblk = pl.num_programs(1)

    an = anch_ref[0]
    ay1 = an[:, 0:1]
    ax1 = an[:, 1:2]
    ay2 = an[:, 2:3]
    ax2 = an[:, 3:4]
    aw = ax2 - ax1
    ah = ay2 - ay1
    area_a = ah * aw

    bt = boxes_ref[0]
    bx1 = bt[0:1, :]
    by1 = bt[1:2, :]
    bx2 = bt[2:3, :]
    by2 = bt[3:4, :]
    lab_row = labs_ref[0]
    valid = lab_row != 0
    has_keep = jnp.any(valid)

    iw = jnp.maximum(jnp.minimum(ax2, bx2) - jnp.maximum(ax1, bx1), 0.0)
    ih = jnp.maximum(jnp.minimum(ay2, by2) - jnp.maximum(ay1, by1), 0.0)
    inter = iw * ih
    area_b = (bx2 - bx1) * (by2 - by1)
    ua = jnp.maximum(area_a + area_b - inter, 1e-8)
    iou = jnp.where(valid, inter / ua, -1.0)
    iou_max = jnp.max(iou, axis=1, keepdims=True)
    nidx = lax.broadcasted_iota(jnp.int32, iou.shape, 1)
    first = jnp.min(jnp.where(iou == iou_max, nidx, iou.shape[1]),
                    axis=1, keepdims=True)
    sel = nidx == first

    def gather(v):
        return jnp.sum(jnp.where(sel, v, 0.0), axis=1, keepdims=True)

    gx1 = gather(bx1)
    gy1 = gather(by1)
    gx2 = gather(bx2)
    gy2 = gather(by2)
    glab = gather((lab_row - 1).astype(jnp.float32))
    big = (gx2 - gx1) * (gy2 - gy1) > 100.0
    pos = (big & (iou_max >= 0.5)) | ((~big) & (iou_max >= 0.15))
    posf = jnp.where(pos, 1.0, 0.0)
    npos = jnp.sum(posf)

    gw_raw = gx2 - gx1
    gh_raw = gy2 - gy1
    gcx = gx1 + 0.5 * gw_raw
    gcy = gy1 + 0.5 * gh_raw
    gw = jnp.maximum(gw_raw, 1.0)
    gh = jnp.maximum(gh_raw, 1.0)
    acx = ax1 + 0.5 * aw
    acy = ay1 + 0.5 * ah
    tdy = (gcy - acy) / ah
    tdx = (gcx - acx) / aw
    tdh = jnp.log(gh / ah)
    tdw = jnp.log(gw / aw)
    rg = reg_ref[0]

    def smooth_l1(tcol, k):
        d = jnp.abs(tcol - rg[:, k:k + 1])
        return jnp.where(d <= 1.0 / 9.0, 4.5 * d * d, d - 0.5 / 9.0)

    rl = smooth_l1(tdy, 0) + smooth_l1(tdx, 1) + smooth_l1(tdh, 2) \
        + smooth_l1(tdw, 3)
    reg_sum = jnp.sum(rl * posf)

    p = jnp.clip(cls_ref[0], 0.0001, 1.0 - 0.0001)
    e = 0.75 * p * p * (-jnp.log(1.0 - p))
    s_empty = jnp.sum(e)
    cidx = lax.broadcasted_iota(jnp.int32, p.shape, 1)
    m = cidx == glab.astype(jnp.int32)
    p1 = jnp.sum(jnp.where(m, p, 0.0), axis=1, keepdims=True)
    p1s = jnp.where(pos, p1, 0.5)
    e1 = 0.75 * p1s * p1s * (-jnp.log(1.0 - p1s))
    f1 = 0.25 * (1.0 - p1s) * (1.0 - p1s) * (-jnp.log(p1s))
    corr = jnp.sum((e1 - f1) * posf)

    @pl.when(b == 0)
    def _init():
        acc_ref[j, 0] = s_empty
        acc_ref[j, 1] = corr
        acc_ref[j, 2] = npos
        acc_ref[j, 3] = reg_sum
        acc_ref[j, 4] = jnp.where(has_keep, 1.0, 0.0)

    @pl.when(b != 0)
    def _accum():
        acc_ref[j, 0] = acc_ref[j, 0] + s_empty
        acc_ref[j, 1] = acc_ref[j, 1] + corr
        acc_ref[j, 2] = acc_ref[j, 2] + npos
        acc_ref[j, 3] = acc_ref[j, 3] + reg_sum

    @pl.when((j == nbatch - 1) & (b == nblk - 1))
    def _final():
        cls_tot = jnp.float32(0.0)
        det_tot = jnp.float32(0.0)
        for jj in range(8):
            se = acc_ref[jj, 0]
            co = acc_ref[jj, 1]
            np_ = acc_ref[jj, 2]
            rs = acc_ref[jj, 3]
            hk = acc_ref[jj, 4]
            cls_full = (se - co) / jnp.maximum(np_, 1.0)
            cls_j = jnp.where(hk > 0.0, cls_full, se)
            reg_j = jnp.where(np_ > 0.0,
                              rs / jnp.maximum(np_ * 4.0, 1.0), 0.0)
            cls_tot = cls_tot + cls_j
            det_tot = det_tot + reg_j
        cls_out[0] = cls_tot / 8.0
        det_out[0] = det_tot / 8.0 * 50.0


def _fused_call(boxes_t, labels3, anchors, regression, classification):
    B, A, C = classification.shape
    nblk = A // _BLKA
    return pl.pallas_call(
        _fused_body,
        grid=(B, nblk),
        in_specs=[
            pl.BlockSpec((1, 4, boxes_t.shape[2]), lambda j, b: (j, 0, 0)),
            pl.BlockSpec((1, 1, labels3.shape[2]), lambda j, b: (j, 0, 0)),
            pl.BlockSpec((1, _BLKA, 4), lambda j, b: (0, b, 0)),
            pl.BlockSpec((1, _BLKA, 4), lambda j, b: (j, b, 0)),
            pl.BlockSpec((1, _BLKA, C), lambda j, b: (j, b, 0)),
        ],
        out_specs=[
            pl.BlockSpec(memory_space=pltpu.SMEM),
            pl.BlockSpec(memory_space=pltpu.SMEM),
        ],
        out_shape=[
            jax.ShapeDtypeStruct((1,), jnp.float32),
            jax.ShapeDtypeStruct((1,), jnp.float32),
        ],
        scratch_shapes=[pltpu.SMEM((B, 5), jnp.float32)],
    )(boxes_t, labels3, anchors, regression, classification)


def kernel(detection_boxes, detection_labels, regression, classification,
           anchors):
    B = classification.shape[0]
    boxes_t = jnp.transpose(detection_boxes, (0, 2, 1))
    labels3 = detection_labels.astype(jnp.int32).reshape(B, 1, -1)
    cls_loss, det_loss = _fused_call(boxes_t, labels3, anchors,
                                     regression, classification)
    return (cls_loss, det_loss)


# speedup vs baseline: 2.1028x; 1.5148x over previous
import jax
import jax.numpy as jnp
from jax import lax
from jax.experimental import pallas as pl
from jax.experimental.pallas import tpu as pltpu

_BLKA = 2000
_AP = 20480
_G = _AP // 128


def _assign_body(boxes_ref, labs_ref, anch_ref, reg_ref,
                 posf_ref, glab_ref, stats_ref):
    nb = boxes_ref.shape[1]
    ay1 = anch_ref[0]
    ax1 = anch_ref[1]
    ay2 = anch_ref[2]
    ax2 = anch_ref[3]
    aw = ax2 - ax1
    ah = ay2 - ay1
    area_a = ah * aw

    best = jnp.full(ax1.shape, -2.0, jnp.float32)
    gx1 = jnp.zeros(ax1.shape, jnp.float32)
    gy1 = gx1
    gx2 = gx1
    gy2 = gx1
    glabf = gx1
    hk = jnp.int32(0)
    for n in range(nb):
        bx1 = boxes_ref[0, n, 0]
        by1 = boxes_ref[0, n, 1]
        bx2 = boxes_ref[0, n, 2]
        by2 = boxes_ref[0, n, 3]
        labn = labs_ref[0, 0, n]
        validn = labn != 0
        hk = hk | jnp.where(validn, 1, 0)
        iw = jnp.maximum(jnp.minimum(ax2, bx2) - jnp.maximum(ax1, bx1), 0.0)
        ih = jnp.maximum(jnp.minimum(ay2, by2) - jnp.maximum(ay1, by1), 0.0)
        inter = iw * ih
        area_b = (bx2 - bx1) * (by2 - by1)
        ua = jnp.maximum(area_a + area_b - inter, 1e-8)
        iou = inter / ua
        iou = jnp.where(validn, iou, -1.0)
        upd = iou > best
        best = jnp.where(upd, iou, best)
        gx1 = jnp.where(upd, bx1, gx1)
        gy1 = jnp.where(upd, by1, gy1)
        gx2 = jnp.where(upd, bx2, gx2)
        gy2 = jnp.where(upd, by2, gy2)
        glabf = jnp.where(upd, (labn - 1).astype(jnp.float32), glabf)

    big = (gx2 - gx1) * (gy2 - gy1) > 100.0
    pos = (big & (best >= 0.5)) | ((~big) & (best >= 0.15))
    aidx = lax.broadcasted_iota(jnp.int32, ax1.shape, 0) * 128 \
        + lax.broadcasted_iota(jnp.int32, ax1.shape, 1)
    posf = jnp.where(pos & (aidx < 20000), 1.0, 0.0)
    npos = jnp.sum(posf)

    gw_raw = gx2 - gx1
    gh_raw = gy2 - gy1
    gcx = gx1 + 0.5 * gw_raw
    gcy = gy1 + 0.5 * gh_raw
    gw = jnp.maximum(gw_raw, 1.0)
    gh = jnp.maximum(gh_raw, 1.0)
    acx = ax1 + 0.5 * aw
    acy = ay1 + 0.5 * ah
    tdy = (gcy - acy) / ah
    tdx = (gcx - acx) / aw
    tdh = jnp.log(gh / ah)
    tdw = jnp.log(gw / aw)

    def smooth_l1(tcol, k):
        d = jnp.abs(tcol - reg_ref[0, k])
        return jnp.where(d <= 1.0 / 9.0, 4.5 * d * d, d - 0.5 / 9.0)

    rl = smooth_l1(tdy, 0) + smooth_l1(tdx, 1) + smooth_l1(tdh, 2) \
        + smooth_l1(tdw, 3)
    reg_sum = jnp.sum(rl * posf)

    posf_ref[0] = posf
    glab_ref[0] = glabf.astype(jnp.int32)
    stats_ref[0, 0, 0] = npos
    stats_ref[0, 0, 1] = reg_sum
    stats_ref[0, 0, 2] = jnp.where(hk > 0, 1.0, 0.0)


def _cls_body(cls_ref, glab_ref, pos_ref, p1_ref, se_ref):
    j = pl.program_id(0)
    b = pl.program_id(1)
    p = jnp.clip(cls_ref[0], 0.0001, 1.0 - 0.0001)
    lg = jnp.log(1.0 - p)
    s_blk = -0.75 * jnp.sum(p * p * lg)
    glabi = glab_ref[0, 0]
    cidx = lax.broadcasted_iota(jnp.int32, p.shape, 1)
    pm = jnp.where(cidx == glabi, p, 0.0)
    p1_ref[0, 0] = jnp.sum(pm, axis=1, keepdims=True)

    @pl.when(b == 0)
    def _():
        se_ref[j] = s_blk

    @pl.when(b != 0)
    def _():
        se_ref[j] = se_ref[j] + s_blk


def _combine_body(p1_ref, posf_ref, stats_ref, se_ref,
                  cls_out, det_out, acc_ref):
    j = pl.program_id(0)
    nbatch = pl.num_programs(0)
    p1 = p1_ref[0]
    posf = posf_ref[0]
    p1s = jnp.where(posf > 0.0, p1, 0.5)
    e1 = 0.75 * p1s * p1s * (-jnp.log(1.0 - p1s))
    f1 = 0.25 * (1.0 - p1s) * (1.0 - p1s) * (-jnp.log(p1s))
    acc_ref[j] = jnp.sum((e1 - f1) * posf)

    @pl.when(j == nbatch - 1)
    def _():
        cls_tot = jnp.float32(0.0)
        det_tot = jnp.float32(0.0)
        for jj in range(8):
            se = se_ref[jj]
            co = acc_ref[jj]
            np_ = stats_ref[jj, 0, 0]
            rs = stats_ref[jj, 0, 1]
            hk = stats_ref[jj, 0, 2]
            cls_full = (se - co) / jnp.maximum(np_, 1.0)
            cls_j = jnp.where(hk > 0.0, cls_full, se)
            reg_j = jnp.where(np_ > 0.0,
                              rs / jnp.maximum(np_ * 4.0, 1.0), 0.0)
            cls_tot = cls_tot + cls_j
            det_tot = det_tot + reg_j
        cls_out[0] = cls_tot / 8.0
        det_out[0] = det_tot / 8.0 * 50.0


def kernel(detection_boxes, detection_labels, regression, classification,
           anchors):
    B, A, C = classification.shape
    nb = detection_boxes.shape[1]
    nblk = A // _BLKA
    padn = _AP - A

    labels = detection_labels.astype(jnp.int32).reshape(B, 1, nb)
    at = jnp.transpose(anchors[0], (1, 0))
    padblk = jnp.tile(jnp.array([[0.0], [0.0], [1.0], [1.0]],
                                jnp.float32), (1, padn))
    at_p = jnp.concatenate([at, padblk], axis=1).reshape(4, _G, 128)
    rt = jnp.transpose(regression, (0, 2, 1))
    rt_p = jnp.pad(rt, ((0, 0), (0, 0), (0, padn))).reshape(B, 4, _G, 128)

    posf_p, glab_p, stats = pl.pallas_call(
        _assign_body,
        grid=(B,),
        in_specs=[
            pl.BlockSpec((1, nb, 4), lambda j: (j, 0, 0),
                         memory_space=pltpu.SMEM),
            pl.BlockSpec((1, 1, nb), lambda j: (j, 0, 0),
                         memory_space=pltpu.SMEM),
            pl.BlockSpec((4, _G, 128), lambda j: (0, 0, 0)),
            pl.BlockSpec((1, 4, _G, 128), lambda j: (j, 0, 0, 0)),
        ],
        out_specs=[
            pl.BlockSpec((1, _G, 128), lambda j: (j, 0, 0)),
            pl.BlockSpec((1, _G, 128), lambda j: (j, 0, 0)),
            pl.BlockSpec((1, 1, 3), lambda j: (j, 0, 0),
                         memory_space=pltpu.SMEM),
        ],
        out_shape=[
            jax.ShapeDtypeStruct((B, _G, 128), jnp.float32),
            jax.ShapeDtypeStruct((B, _G, 128), jnp.int32),
            jax.ShapeDtypeStruct((B, 1, 3), jnp.float32),
        ],
    )(detection_boxes, labels, at_p, rt_p)

    posf_a = posf_p.reshape(B, _AP)[:, :A]
    glab_a = glab_p.reshape(B, _AP)[:, :A]
    pos_b = posf_a.reshape(B, nblk, _BLKA, 1)
    glab_b = glab_a.reshape(B, nblk, _BLKA, 1)

    p1_b, se = pl.pallas_call(
        _cls_body,
        grid=(B, nblk),
        in_specs=[
            pl.BlockSpec((1, _BLKA, C), lambda j, b: (j, b, 0)),
            pl.BlockSpec((1, 1, _BLKA, 1), lambda j, b: (j, b, 0, 0)),
            pl.BlockSpec((1, 1, _BLKA, 1), lambda j, b: (j, b, 0, 0)),
        ],
        out_specs=[
            pl.BlockSpec((1, 1, _BLKA, 1), lambda j, b: (j, b, 0, 0)),
            pl.BlockSpec(memory_space=pltpu.SMEM),
        ],
        out_shape=[
            jax.ShapeDtypeStruct((B, nblk, _BLKA, 1), jnp.float32),
            jax.ShapeDtypeStruct((B,), jnp.float32),
        ],
    )(classification, glab_b, pos_b)

    p1_d = jnp.pad(p1_b.reshape(B, A), ((0, 0), (0, padn))) \
        .reshape(B, _G, 128)

    cls_loss, det_loss = pl.pallas_call(
        _combine_body,
        grid=(B,),
        in_specs=[
            pl.BlockSpec((1, _G, 128), lambda j: (j, 0, 0)),
            pl.BlockSpec((1, _G, 128), lambda j: (j, 0, 0)),
            pl.BlockSpec(memory_space=pltpu.SMEM),
            pl.BlockSpec(memory_space=pltpu.SMEM),
        ],
        out_specs=[
            pl.BlockSpec(memory_space=pltpu.SMEM),
            pl.BlockSpec(memory_space=pltpu.SMEM),
        ],
        out_shape=[
            jax.ShapeDtypeStruct((1,), jnp.float32),
            jax.ShapeDtypeStruct((1,), jnp.float32),
        ],
        scratch_shapes=[pltpu.SMEM((B,), jnp.float32)],
    )(p1_d, posf_p, stats, se)

    return (cls_loss, det_loss)


# speedup vs baseline: 4.7491x; 2.2585x over previous
import jax
import jax.numpy as jnp
from jax import lax
from jax.experimental import pallas as pl
from jax.experimental.pallas import tpu as pltpu

_BLKA = 2000
_AP = 20480
_G = _AP // 128


def _assign_body(boxes_ref, labs_ref, anch_ref, reg_ref,
                 posf_ref, glab_ref, stats_ref):
    nb = boxes_ref.shape[1]
    ay1 = anch_ref[0]
    ax1 = anch_ref[1]
    ay2 = anch_ref[2]
    ax2 = anch_ref[3]
    aw = ax2 - ax1
    ah = ay2 - ay1
    area_a = ah * aw

    best = jnp.full(ax1.shape, -2.0, jnp.float32)
    gx1 = jnp.zeros(ax1.shape, jnp.float32)
    gy1 = gx1
    gx2 = gx1
    gy2 = gx1
    glabf = gx1
    hk = jnp.int32(0)
    for n in range(nb):
        bx1 = boxes_ref[0, n, 0]
        by1 = boxes_ref[0, n, 1]
        bx2 = boxes_ref[0, n, 2]
        by2 = boxes_ref[0, n, 3]
        labn = labs_ref[0, 0, n]
        validn = labn != 0
        hk = hk | jnp.where(validn, 1, 0)
        iw = jnp.maximum(jnp.minimum(ax2, bx2) - jnp.maximum(ax1, bx1), 0.0)
        ih = jnp.maximum(jnp.minimum(ay2, by2) - jnp.maximum(ay1, by1), 0.0)
        inter = iw * ih
        area_b = (bx2 - bx1) * (by2 - by1)
        ua = jnp.maximum(area_a + area_b - inter, 1e-8)
        iou = inter / ua
        iou = jnp.where(validn, iou, -1.0)
        upd = iou > best
        best = jnp.where(upd, iou, best)
        gx1 = jnp.where(upd, bx1, gx1)
        gy1 = jnp.where(upd, by1, gy1)
        gx2 = jnp.where(upd, bx2, gx2)
        gy2 = jnp.where(upd, by2, gy2)
        glabf = jnp.where(upd, (labn - 1).astype(jnp.float32), glabf)

    big = (gx2 - gx1) * (gy2 - gy1) > 100.0
    pos = (big & (best >= 0.5)) | ((~big) & (best >= 0.15))
    aidx = lax.broadcasted_iota(jnp.int32, ax1.shape, 0) * 128 \
        + lax.broadcasted_iota(jnp.int32, ax1.shape, 1)
    posf = jnp.where(pos & (aidx < 20000), 1.0, 0.0)
    npos = jnp.sum(posf)

    gw_raw = gx2 - gx1
    gh_raw = gy2 - gy1
    gcx = gx1 + 0.5 * gw_raw
    gcy = gy1 + 0.5 * gh_raw
    gw = jnp.maximum(gw_raw, 1.0)
    gh = jnp.maximum(gh_raw, 1.0)
    acx = ax1 + 0.5 * aw
    acy = ay1 + 0.5 * ah
    tdy = (gcy - acy) / ah
    tdx = (gcx - acx) / aw
    tdh = jnp.log(gh / ah)
    tdw = jnp.log(gw / aw)

    def smooth_l1(tcol, k):
        d = jnp.abs(tcol - reg_ref[0, k])
        return jnp.where(d <= 1.0 / 9.0, 4.5 * d * d, d - 0.5 / 9.0)

    rl = smooth_l1(tdy, 0) + smooth_l1(tdx, 1) + smooth_l1(tdh, 2) \
        + smooth_l1(tdw, 3)
    reg_sum = jnp.sum(rl * posf)

    posf_ref[0] = posf
    glab_ref[0] = glabf
    stats_ref[0, 0, 0] = npos
    stats_ref[0, 0, 1] = reg_sum
    stats_ref[0, 0, 2] = jnp.where(hk > 0, 1.0, 0.0)


def _cls_body(cls_ref, glab_ref, pos_ref, stats_ref,
              cls_out, det_out, acc_ref):
    j = pl.program_id(0)
    b = pl.program_id(1)
    nbatch = pl.num_programs(0)
    nblk = pl.num_programs(1)

    p = jnp.clip(cls_ref[0], 0.0001, 1.0 - 0.0001)
    lg = jnp.log(1.0 - p)
    s_blk = -0.75 * jnp.sum(p * p * lg)

    glab_row = glab_ref[0, 0].astype(jnp.int32)
    glab_col = jnp.transpose(glab_row, (1, 0))
    cidx = lax.broadcasted_iota(jnp.int32, p.shape, 1)
    pm = jnp.where(cidx == glab_col, p, 0.0)
    p1_row = lax.dot_general(
        jnp.ones((1, p.shape[1]), jnp.float32), pm,
        (((1,), (1,)), ((), ())),
        precision=lax.Precision.HIGHEST,
        preferred_element_type=jnp.float32)
    posf_row = pos_ref[0, 0]
    p1s = jnp.where(posf_row > 0.0, p1_row, 0.5)
    e1 = 0.75 * p1s * p1s * (-jnp.log(1.0 - p1s))
    f1 = 0.25 * (1.0 - p1s) * (1.0 - p1s) * (-jnp.log(p1s))
    corr_blk = jnp.sum((e1 - f1) * posf_row)

    @pl.when(b == 0)
    def _():
        acc_ref[j, 0] = s_blk
        acc_ref[j, 1] = corr_blk

    @pl.when(b != 0)
    def _():
        acc_ref[j, 0] = acc_ref[j, 0] + s_blk
        acc_ref[j, 1] = acc_ref[j, 1] + corr_blk

    @pl.when((j == nbatch - 1) & (b == nblk - 1))
    def _():
        cls_tot = jnp.float32(0.0)
        det_tot = jnp.float32(0.0)
        for jj in range(8):
            se = acc_ref[jj, 0]
            co = acc_ref[jj, 1]
            np_ = stats_ref[jj, 0, 0]
            rs = stats_ref[jj, 0, 1]
            hk = stats_ref[jj, 0, 2]
            cls_full = (se - co) / jnp.maximum(np_, 1.0)
            cls_j = jnp.where(hk > 0.0, cls_full, se)
            reg_j = jnp.where(np_ > 0.0,
                              rs / jnp.maximum(np_ * 4.0, 1.0), 0.0)
            cls_tot = cls_tot + cls_j
            det_tot = det_tot + reg_j
        cls_out[0] = cls_tot / 8.0
        det_out[0] = det_tot / 8.0 * 50.0


def kernel(detection_boxes, detection_labels, regression, classification,
           anchors):
    B, A, C = classification.shape
    nb = detection_boxes.shape[1]
    nblk = A // _BLKA
    padn = _AP - A

    labels = detection_labels.astype(jnp.int32).reshape(B, 1, nb)
    at = jnp.transpose(anchors[0], (1, 0))
    padblk = jnp.tile(jnp.array([[0.0], [0.0], [1.0], [1.0]],
                                jnp.float32), (1, padn))
    at_p = jnp.concatenate([at, padblk], axis=1).reshape(4, _G, 128)
    rt = jnp.transpose(regression, (0, 2, 1))
    rt_p = jnp.pad(rt, ((0, 0), (0, 0), (0, padn))).reshape(B, 4, _G, 128)

    posf_p, glab_p, stats = pl.pallas_call(
        _assign_body,
        grid=(B,),
        in_specs=[
            pl.BlockSpec((1, nb, 4), lambda j: (j, 0, 0),
                         memory_space=pltpu.SMEM),
            pl.BlockSpec((1, 1, nb), lambda j: (j, 0, 0),
                         memory_space=pltpu.SMEM),
            pl.BlockSpec((4, _G, 128), lambda j: (0, 0, 0)),
            pl.BlockSpec((1, 4, _G, 128), lambda j: (j, 0, 0, 0)),
        ],
        out_specs=[
            pl.BlockSpec((1, _G, 128), lambda j: (j, 0, 0)),
            pl.BlockSpec((1, _G, 128), lambda j: (j, 0, 0)),
            pl.BlockSpec((1, 1, 3), lambda j: (j, 0, 0),
                         memory_space=pltpu.SMEM),
        ],
        out_shape=[
            jax.ShapeDtypeStruct((B, _G, 128), jnp.float32),
            jax.ShapeDtypeStruct((B, _G, 128), jnp.float32),
            jax.ShapeDtypeStruct((B, 1, 3), jnp.float32),
        ],
    )(detection_boxes, labels, at_p, rt_p)

    posf_r = posf_p.reshape(B, _AP)[:, :A].reshape(B, nblk, 1, _BLKA)
    glab_r = glab_p.reshape(B, _AP)[:, :A].reshape(B, nblk, 1, _BLKA)

    cls_loss, det_loss = pl.pallas_call(
        _cls_body,
        grid=(B, nblk),
        in_specs=[
            pl.BlockSpec((1, _BLKA, C), lambda j, b: (j, b, 0)),
            pl.BlockSpec((1, 1, 1, _BLKA), lambda j, b: (j, b, 0, 0)),
            pl.BlockSpec((1, 1, 1, _BLKA), lambda j, b: (j, b, 0, 0)),
            pl.BlockSpec(memory_space=pltpu.SMEM),
        ],
        out_specs=[
            pl.BlockSpec(memory_space=pltpu.SMEM),
            pl.BlockSpec(memory_space=pltpu.SMEM),
        ],
        out_shape=[
            jax.ShapeDtypeStruct((1,), jnp.float32),
            jax.ShapeDtypeStruct((1,), jnp.float32),
        ],
        scratch_shapes=[pltpu.SMEM((B, 2), jnp.float32)],
    )(classification, glab_r, posf_r, stats)

    return (cls_loss, det_loss)


# speedup vs baseline: 5.1127x; 1.0766x over previous
import jax
import jax.numpy as jnp
from jax import lax
from jax.experimental import pallas as pl
from jax.experimental.pallas import tpu as pltpu

_BLKA = 4000
_AP = 20480
_G = _AP // 128


def _assign_body(boxes_ref, labs_ref, anch_ref, reg_ref,
                 posf_ref, glab_ref, stats_ref):
    nb = boxes_ref.shape[1]
    ay1 = anch_ref[0]
    ax1 = anch_ref[1]
    ay2 = anch_ref[2]
    ax2 = anch_ref[3]
    aw = ax2 - ax1
    ah = ay2 - ay1
    area_a = ah * aw

    best = jnp.full(ax1.shape, -2.0, jnp.float32)
    gx1 = jnp.zeros(ax1.shape, jnp.float32)
    gy1 = gx1
    gx2 = gx1
    gy2 = gx1
    glabf = gx1
    hk = jnp.int32(0)
    for n in range(nb):
        bx1 = boxes_ref[0, n, 0]
        by1 = boxes_ref[0, n, 1]
        bx2 = boxes_ref[0, n, 2]
        by2 = boxes_ref[0, n, 3]
        labn = labs_ref[0, 0, n]
        validn = labn != 0
        hk = hk | jnp.where(validn, 1, 0)
        iw = jnp.maximum(jnp.minimum(ax2, bx2) - jnp.maximum(ax1, bx1), 0.0)
        ih = jnp.maximum(jnp.minimum(ay2, by2) - jnp.maximum(ay1, by1), 0.0)
        inter = iw * ih
        area_b = (bx2 - bx1) * (by2 - by1)
        ua = jnp.maximum(area_a + area_b - inter, 1e-8)
        iou = inter / ua
        iou = jnp.where(validn, iou, -1.0)
        upd = iou > best
        best = jnp.where(upd, iou, best)
        gx1 = jnp.where(upd, bx1, gx1)
        gy1 = jnp.where(upd, by1, gy1)
        gx2 = jnp.where(upd, bx2, gx2)
        gy2 = jnp.where(upd, by2, gy2)
        glabf = jnp.where(upd, (labn - 1).astype(jnp.float32), glabf)

    big = (gx2 - gx1) * (gy2 - gy1) > 100.0
    pos = (big & (best >= 0.5)) | ((~big) & (best >= 0.15))
    aidx = lax.broadcasted_iota(jnp.int32, ax1.shape, 0) * 128 \
        + lax.broadcasted_iota(jnp.int32, ax1.shape, 1)
    posf = jnp.where(pos & (aidx < 20000), 1.0, 0.0)
    npos = jnp.sum(posf)

    gw_raw = gx2 - gx1
    gh_raw = gy2 - gy1
    gcx = gx1 + 0.5 * gw_raw
    gcy = gy1 + 0.5 * gh_raw
    gw = jnp.maximum(gw_raw, 1.0)
    gh = jnp.maximum(gh_raw, 1.0)
    acx = ax1 + 0.5 * aw
    acy = ay1 + 0.5 * ah
    tdy = (gcy - acy) / ah
    tdx = (gcx - acx) / aw
    tdh = jnp.log(gh / ah)
    tdw = jnp.log(gw / aw)

    def smooth_l1(tcol, k):
        d = jnp.abs(tcol - reg_ref[0, k])
        return jnp.where(d <= 1.0 / 9.0, 4.5 * d * d, d - 0.5 / 9.0)

    rl = smooth_l1(tdy, 0) + smooth_l1(tdx, 1) + smooth_l1(tdh, 2) \
        + smooth_l1(tdw, 3)
    reg_sum = jnp.sum(rl * posf)

    posf_ref[0] = posf
    glab_ref[0] = glabf
    stats_ref[0, 0, 0] = npos
    stats_ref[0, 0, 1] = reg_sum
    stats_ref[0, 0, 2] = jnp.where(hk > 0, 1.0, 0.0)


def _cls_body(cls_ref, glab_ref, pos_ref, stats_ref,
              cls_out, det_out, acc_ref):
    j = pl.program_id(0)
    b = pl.program_id(1)
    nbatch = pl.num_programs(0)
    nblk = pl.num_programs(1)

    p = jnp.clip(cls_ref[0], 0.0001, 1.0 - 0.0001)
    lg = jnp.log(1.0 - p)
    s_blk = -0.75 * jnp.sum(p * p * lg)

    glab_row = glab_ref[0, 0].astype(jnp.int32)
    glab_col = jnp.transpose(glab_row, (1, 0))
    cidx = lax.broadcasted_iota(jnp.int32, p.shape, 1)
    pm = jnp.where(cidx == glab_col, p, 0.0)
    p1_row = lax.dot_general(
        jnp.ones((1, p.shape[1]), jnp.float32), pm,
        (((1,), (1,)), ((), ())),
        precision=lax.Precision.HIGHEST,
        preferred_element_type=jnp.float32)
    posf_row = pos_ref[0, 0]
    p1s = jnp.where(posf_row > 0.0, p1_row, 0.5)
    e1 = 0.75 * p1s * p1s * (-jnp.log(1.0 - p1s))
    f1 = 0.25 * (1.0 - p1s) * (1.0 - p1s) * (-jnp.log(p1s))
    corr_blk = jnp.sum((e1 - f1) * posf_row)

    @pl.when(b == 0)
    def _():
        acc_ref[j, 0] = s_blk
        acc_ref[j, 1] = corr_blk

    @pl.when(b != 0)
    def _():
        acc_ref[j, 0] = acc_ref[j, 0] + s_blk
        acc_ref[j, 1] = acc_ref[j, 1] + corr_blk

    @pl.when((j == nbatch - 1) & (b == nblk - 1))
    def _():
        cls_tot = jnp.float32(0.0)
        det_tot = jnp.float32(0.0)
        for jj in range(8):
            se = acc_ref[jj, 0]
            co = acc_ref[jj, 1]
            np_ = stats_ref[jj, 0, 0]
            rs = stats_ref[jj, 0, 1]
            hk = stats_ref[jj, 0, 2]
            cls_full = (se - co) / jnp.maximum(np_, 1.0)
            cls_j = jnp.where(hk > 0.0, cls_full, se)
            reg_j = jnp.where(np_ > 0.0,
                              rs / jnp.maximum(np_ * 4.0, 1.0), 0.0)
            cls_tot = cls_tot + cls_j
            det_tot = det_tot + reg_j
        cls_out[0] = cls_tot / 8.0
        det_out[0] = det_tot / 8.0 * 50.0


def kernel(detection_boxes, detection_labels, regression, classification,
           anchors):
    B, A, C = classification.shape
    nb = detection_boxes.shape[1]
    nblk = A // _BLKA
    padn = _AP - A

    labels = detection_labels.astype(jnp.int32).reshape(B, 1, nb)
    at = jnp.transpose(anchors[0], (1, 0))
    padblk = jnp.tile(jnp.array([[0.0], [0.0], [1.0], [1.0]],
                                jnp.float32), (1, padn))
    at_p = jnp.concatenate([at, padblk], axis=1).reshape(4, _G, 128)
    rt = jnp.transpose(regression, (0, 2, 1))
    rt_p = jnp.pad(rt, ((0, 0), (0, 0), (0, padn))).reshape(B, 4, _G, 128)

    posf_p, glab_p, stats = pl.pallas_call(
        _assign_body,
        grid=(B,),
        in_specs=[
            pl.BlockSpec((1, nb, 4), lambda j: (j, 0, 0),
                         memory_space=pltpu.SMEM),
            pl.BlockSpec((1, 1, nb), lambda j: (j, 0, 0),
                         memory_space=pltpu.SMEM),
            pl.BlockSpec((4, _G, 128), lambda j: (0, 0, 0)),
            pl.BlockSpec((1, 4, _G, 128), lambda j: (j, 0, 0, 0)),
        ],
        out_specs=[
            pl.BlockSpec((1, _G, 128), lambda j: (j, 0, 0)),
            pl.BlockSpec((1, _G, 128), lambda j: (j, 0, 0)),
            pl.BlockSpec((1, 1, 3), lambda j: (j, 0, 0),
                         memory_space=pltpu.SMEM),
        ],
        out_shape=[
            jax.ShapeDtypeStruct((B, _G, 128), jnp.float32),
            jax.ShapeDtypeStruct((B, _G, 128), jnp.float32),
            jax.ShapeDtypeStruct((B, 1, 3), jnp.float32),
        ],
    )(detection_boxes, labels, at_p, rt_p)

    posf_r = posf_p.reshape(B, _AP)[:, :A].reshape(B, nblk, 1, _BLKA)
    glab_r = glab_p.reshape(B, _AP)[:, :A].reshape(B, nblk, 1, _BLKA)

    cls_loss, det_loss = pl.pallas_call(
        _cls_body,
        grid=(B, nblk),
        in_specs=[
            pl.BlockSpec((1, _BLKA, C), lambda j, b: (j, b, 0)),
            pl.BlockSpec((1, 1, 1, _BLKA), lambda j, b: (j, b, 0, 0)),
            pl.BlockSpec((1, 1, 1, _BLKA), lambda j, b: (j, b, 0, 0)),
            pl.BlockSpec(memory_space=pltpu.SMEM),
        ],
        out_specs=[
            pl.BlockSpec(memory_space=pltpu.SMEM),
            pl.BlockSpec(memory_space=pltpu.SMEM),
        ],
        out_shape=[
            jax.ShapeDtypeStruct((1,), jnp.float32),
            jax.ShapeDtypeStruct((1,), jnp.float32),
        ],
        scratch_shapes=[pltpu.SMEM((B, 2), jnp.float32)],
    )(classification, glab_r, posf_r, stats)

    return (cls_loss, det_loss)


# speedup vs baseline: 5.7351x; 1.1217x over previous
import jax
import jax.numpy as jnp
from jax import lax
from jax.experimental import pallas as pl
from jax.experimental.pallas import tpu as pltpu

_BLKA = 10000
_AP = 20480
_G = _AP // 128


def _assign_body(boxes_ref, labs_ref, anch_ref, reg_ref,
                 posf_ref, glab_ref, stats_ref):
    nb = boxes_ref.shape[1]
    ay1 = anch_ref[0]
    ax1 = anch_ref[1]
    ay2 = anch_ref[2]
    ax2 = anch_ref[3]
    aw = ax2 - ax1
    ah = ay2 - ay1
    area_a = ah * aw

    best = jnp.full(ax1.shape, -2.0, jnp.float32)
    gx1 = jnp.zeros(ax1.shape, jnp.float32)
    gy1 = gx1
    gx2 = gx1
    gy2 = gx1
    glabf = gx1
    hk = jnp.int32(0)
    for n in range(nb):
        bx1 = boxes_ref[0, n, 0]
        by1 = boxes_ref[0, n, 1]
        bx2 = boxes_ref[0, n, 2]
        by2 = boxes_ref[0, n, 3]
        labn = labs_ref[0, 0, n]
        validn = labn != 0
        hk = hk | jnp.where(validn, 1, 0)
        iw = jnp.maximum(jnp.minimum(ax2, bx2) - jnp.maximum(ax1, bx1), 0.0)
        ih = jnp.maximum(jnp.minimum(ay2, by2) - jnp.maximum(ay1, by1), 0.0)
        inter = iw * ih
        area_b = (bx2 - bx1) * (by2 - by1)
        ua = jnp.maximum(area_a + area_b - inter, 1e-8)
        iou = inter / ua
        iou = jnp.where(validn, iou, -1.0)
        upd = iou > best
        best = jnp.where(upd, iou, best)
        gx1 = jnp.where(upd, bx1, gx1)
        gy1 = jnp.where(upd, by1, gy1)
        gx2 = jnp.where(upd, bx2, gx2)
        gy2 = jnp.where(upd, by2, gy2)
        glabf = jnp.where(upd, (labn - 1).astype(jnp.float32), glabf)

    big = (gx2 - gx1) * (gy2 - gy1) > 100.0
    pos = (big & (best >= 0.5)) | ((~big) & (best >= 0.15))
    aidx = lax.broadcasted_iota(jnp.int32, ax1.shape, 0) * 128 \
        + lax.broadcasted_iota(jnp.int32, ax1.shape, 1)
    posf = jnp.where(pos & (aidx < 20000), 1.0, 0.0)
    npos = jnp.sum(posf)

    gw_raw = gx2 - gx1
    gh_raw = gy2 - gy1
    gcx = gx1 + 0.5 * gw_raw
    gcy = gy1 + 0.5 * gh_raw
    gw = jnp.maximum(gw_raw, 1.0)
    gh = jnp.maximum(gh_raw, 1.0)
    acx = ax1 + 0.5 * aw
    acy = ay1 + 0.5 * ah
    tdy = (gcy - acy) / ah
    tdx = (gcx - acx) / aw
    tdh = jnp.log(gh / ah)
    tdw = jnp.log(gw / aw)

    def smooth_l1(tcol, k):
        d = jnp.abs(tcol - reg_ref[0, k])
        return jnp.where(d <= 1.0 / 9.0, 4.5 * d * d, d - 0.5 / 9.0)

    rl = smooth_l1(tdy, 0) + smooth_l1(tdx, 1) + smooth_l1(tdh, 2) \
        + smooth_l1(tdw, 3)
    reg_sum = jnp.sum(rl * posf)

    posf_ref[0] = posf
    glab_ref[0] = glabf
    stats_ref[0, 0, 0] = npos
    stats_ref[0, 0, 1] = reg_sum
    stats_ref[0, 0, 2] = jnp.where(hk > 0, 1.0, 0.0)


def _cls_body(cls_ref, glab_ref, pos_ref, stats_ref,
              cls_out, det_out, acc_ref):
    j = pl.program_id(0)
    b = pl.program_id(1)
    nbatch = pl.num_programs(0)
    nblk = pl.num_programs(1)

    p = jnp.clip(cls_ref[0], 0.0001, 1.0 - 0.0001)
    lg = jnp.log(1.0 - p)
    s_blk = -0.75 * jnp.sum(p * p * lg)

    glab_row = glab_ref[0, 0].astype(jnp.int32)
    glab_col = jnp.transpose(glab_row, (1, 0))
    cidx = lax.broadcasted_iota(jnp.int32, p.shape, 1)
    pm = jnp.where(cidx == glab_col, p, 0.0)
    p1_row = lax.dot_general(
        jnp.ones((1, p.shape[1]), jnp.float32), pm,
        (((1,), (1,)), ((), ())),
        precision=lax.Precision.HIGHEST,
        preferred_element_type=jnp.float32)
    posf_row = pos_ref[0, 0]
    p1s = jnp.where(posf_row > 0.0, p1_row, 0.5)
    e1 = 0.75 * p1s * p1s * (-jnp.log(1.0 - p1s))
    f1 = 0.25 * (1.0 - p1s) * (1.0 - p1s) * (-jnp.log(p1s))
    corr_blk = jnp.sum((e1 - f1) * posf_row)

    @pl.when(b == 0)
    def _():
        acc_ref[j, 0] = s_blk
        acc_ref[j, 1] = corr_blk

    @pl.when(b != 0)
    def _():
        acc_ref[j, 0] = acc_ref[j, 0] + s_blk
        acc_ref[j, 1] = acc_ref[j, 1] + corr_blk

    @pl.when((j == nbatch - 1) & (b == nblk - 1))
    def _():
        cls_tot = jnp.float32(0.0)
        det_tot = jnp.float32(0.0)
        for jj in range(8):
            se = acc_ref[jj, 0]
            co = acc_ref[jj, 1]
            np_ = stats_ref[jj, 0, 0]
            rs = stats_ref[jj, 0, 1]
            hk = stats_ref[jj, 0, 2]
            cls_full = (se - co) / jnp.maximum(np_, 1.0)
            cls_j = jnp.where(hk > 0.0, cls_full, se)
            reg_j = jnp.where(np_ > 0.0,
                              rs / jnp.maximum(np_ * 4.0, 1.0), 0.0)
            cls_tot = cls_tot + cls_j
            det_tot = det_tot + reg_j
        cls_out[0] = cls_tot / 8.0
        det_out[0] = det_tot / 8.0 * 50.0


def kernel(detection_boxes, detection_labels, regression, classification,
           anchors):
    B, A, C = classification.shape
    nb = detection_boxes.shape[1]
    nblk = A // _BLKA
    padn = _AP - A

    labels = detection_labels.astype(jnp.int32).reshape(B, 1, nb)
    at = jnp.transpose(anchors[0], (1, 0))
    padblk = jnp.tile(jnp.array([[0.0], [0.0], [1.0], [1.0]],
                                jnp.float32), (1, padn))
    at_p = jnp.concatenate([at, padblk], axis=1).reshape(4, _G, 128)
    rt = jnp.transpose(regression, (0, 2, 1))
    rt_p = jnp.pad(rt, ((0, 0), (0, 0), (0, padn))).reshape(B, 4, _G, 128)

    posf_p, glab_p, stats = pl.pallas_call(
        _assign_body,
        grid=(B,),
        in_specs=[
            pl.BlockSpec((1, nb, 4), lambda j: (j, 0, 0),
                         memory_space=pltpu.SMEM),
            pl.BlockSpec((1, 1, nb), lambda j: (j, 0, 0),
                         memory_space=pltpu.SMEM),
            pl.BlockSpec((4, _G, 128), lambda j: (0, 0, 0)),
            pl.BlockSpec((1, 4, _G, 128), lambda j: (j, 0, 0, 0)),
        ],
        out_specs=[
            pl.BlockSpec((1, _G, 128), lambda j: (j, 0, 0)),
            pl.BlockSpec((1, _G, 128), lambda j: (j, 0, 0)),
            pl.BlockSpec((1, 1, 3), lambda j: (j, 0, 0),
                         memory_space=pltpu.SMEM),
        ],
        out_shape=[
            jax.ShapeDtypeStruct((B, _G, 128), jnp.float32),
            jax.ShapeDtypeStruct((B, _G, 128), jnp.float32),
            jax.ShapeDtypeStruct((B, 1, 3), jnp.float32),
        ],
    )(detection_boxes, labels, at_p, rt_p)

    posf_r = posf_p.reshape(B, _AP)[:, :A].reshape(B, nblk, 1, _BLKA)
    glab_r = glab_p.reshape(B, _AP)[:, :A].reshape(B, nblk, 1, _BLKA)

    cls_loss, det_loss = pl.pallas_call(
        _cls_body,
        grid=(B, nblk),
        in_specs=[
            pl.BlockSpec((1, _BLKA, C), lambda j, b: (j, b, 0)),
            pl.BlockSpec((1, 1, 1, _BLKA), lambda j, b: (j, b, 0, 0)),
            pl.BlockSpec((1, 1, 1, _BLKA), lambda j, b: (j, b, 0, 0)),
            pl.BlockSpec(memory_space=pltpu.SMEM),
        ],
        out_specs=[
            pl.BlockSpec(memory_space=pltpu.SMEM),
            pl.BlockSpec(memory_space=pltpu.SMEM),
        ],
        out_shape=[
            jax.ShapeDtypeStruct((1,), jnp.float32),
            jax.ShapeDtypeStruct((1,), jnp.float32),
        ],
        scratch_shapes=[pltpu.SMEM((B, 2), jnp.float32)],
    )(classification, glab_r, posf_r, stats)

    return (cls_loss, det_loss)


# speedup vs baseline: 5.8632x; 1.0223x over previous
import jax
import jax.numpy as jnp
from jax import lax
from jax.experimental import pallas as pl
from jax.experimental.pallas import tpu as pltpu

_BLKA = 20000
_AP = 20480
_G = _AP // 128


def _assign_body(boxes_ref, labs_ref, anch_ref, reg_ref,
                 posf_ref, glab_ref, stats_ref):
    nb = boxes_ref.shape[1]
    ay1 = anch_ref[0]
    ax1 = anch_ref[1]
    ay2 = anch_ref[2]
    ax2 = anch_ref[3]
    aw = ax2 - ax1
    ah = ay2 - ay1
    area_a = ah * aw

    best = jnp.full(ax1.shape, -2.0, jnp.float32)
    gx1 = jnp.zeros(ax1.shape, jnp.float32)
    gy1 = gx1
    gx2 = gx1
    gy2 = gx1
    glabf = gx1
    hk = jnp.int32(0)
    for n in range(nb):
        bx1 = boxes_ref[0, n, 0]
        by1 = boxes_ref[0, n, 1]
        bx2 = boxes_ref[0, n, 2]
        by2 = boxes_ref[0, n, 3]
        labn = labs_ref[0, 0, n]
        validn = labn != 0
        hk = hk | jnp.where(validn, 1, 0)
        iw = jnp.maximum(jnp.minimum(ax2, bx2) - jnp.maximum(ax1, bx1), 0.0)
        ih = jnp.maximum(jnp.minimum(ay2, by2) - jnp.maximum(ay1, by1), 0.0)
        inter = iw * ih
        area_b = (bx2 - bx1) * (by2 - by1)
        ua = jnp.maximum(area_a + area_b - inter, 1e-8)
        iou = inter / ua
        iou = jnp.where(validn, iou, -1.0)
        upd = iou > best
        best = jnp.where(upd, iou, best)
        gx1 = jnp.where(upd, bx1, gx1)
        gy1 = jnp.where(upd, by1, gy1)
        gx2 = jnp.where(upd, bx2, gx2)
        gy2 = jnp.where(upd, by2, gy2)
        glabf = jnp.where(upd, (labn - 1).astype(jnp.float32), glabf)

    big = (gx2 - gx1) * (gy2 - gy1) > 100.0
    pos = (big & (best >= 0.5)) | ((~big) & (best >= 0.15))
    aidx = lax.broadcasted_iota(jnp.int32, ax1.shape, 0) * 128 \
        + lax.broadcasted_iota(jnp.int32, ax1.shape, 1)
    posf = jnp.where(pos & (aidx < 20000), 1.0, 0.0)
    npos = jnp.sum(posf)

    gw_raw = gx2 - gx1
    gh_raw = gy2 - gy1
    gcx = gx1 + 0.5 * gw_raw
    gcy = gy1 + 0.5 * gh_raw
    gw = jnp.maximum(gw_raw, 1.0)
    gh = jnp.maximum(gh_raw, 1.0)
    acx = ax1 + 0.5 * aw
    acy = ay1 + 0.5 * ah
    tdy = (gcy - acy) / ah
    tdx = (gcx - acx) / aw
    tdh = jnp.log(gh / ah)
    tdw = jnp.log(gw / aw)

    def smooth_l1(tcol, k):
        d = jnp.abs(tcol - reg_ref[0, k])
        return jnp.where(d <= 1.0 / 9.0, 4.5 * d * d, d - 0.5 / 9.0)

    rl = smooth_l1(tdy, 0) + smooth_l1(tdx, 1) + smooth_l1(tdh, 2) \
        + smooth_l1(tdw, 3)
    reg_sum = jnp.sum(rl * posf)

    posf_ref[0] = posf
    glab_ref[0] = glabf
    stats_ref[0, 0, 0] = npos
    stats_ref[0, 0, 1] = reg_sum
    stats_ref[0, 0, 2] = jnp.where(hk > 0, 1.0, 0.0)


def _cls_body(cls_ref, glab_ref, pos_ref, stats_ref,
              cls_out, det_out, acc_ref):
    j = pl.program_id(0)
    b = pl.program_id(1)
    nbatch = pl.num_programs(0)
    nblk = pl.num_programs(1)

    p = jnp.clip(cls_ref[0], 0.0001, 1.0 - 0.0001)
    lg = jnp.log(1.0 - p)
    s_blk = -0.75 * jnp.sum(p * p * lg)

    glab_row = glab_ref[0, 0].astype(jnp.int32)
    glab_col = jnp.transpose(glab_row, (1, 0))
    cidx = lax.broadcasted_iota(jnp.int32, p.shape, 1)
    pm = jnp.where(cidx == glab_col, p, 0.0)
    p1_row = lax.dot_general(
        jnp.ones((1, p.shape[1]), jnp.float32), pm,
        (((1,), (1,)), ((), ())),
        precision=lax.Precision.HIGHEST,
        preferred_element_type=jnp.float32)
    posf_row = pos_ref[0, 0]
    p1s = jnp.where(posf_row > 0.0, p1_row, 0.5)
    e1 = 0.75 * p1s * p1s * (-jnp.log(1.0 - p1s))
    f1 = 0.25 * (1.0 - p1s) * (1.0 - p1s) * (-jnp.log(p1s))
    corr_blk = jnp.sum((e1 - f1) * posf_row)

    @pl.when(b == 0)
    def _():
        acc_ref[j, 0] = s_blk
        acc_ref[j, 1] = corr_blk

    @pl.when(b != 0)
    def _():
        acc_ref[j, 0] = acc_ref[j, 0] + s_blk
        acc_ref[j, 1] = acc_ref[j, 1] + corr_blk

    @pl.when((j == nbatch - 1) & (b == nblk - 1))
    def _():
        cls_tot = jnp.float32(0.0)
        det_tot = jnp.float32(0.0)
        for jj in range(8):
            se = acc_ref[jj, 0]
            co = acc_ref[jj, 1]
            np_ = stats_ref[jj, 0, 0]
            rs = stats_ref[jj, 0, 1]
            hk = stats_ref[jj, 0, 2]
            cls_full = (se - co) / jnp.maximum(np_, 1.0)
            cls_j = jnp.where(hk > 0.0, cls_full, se)
            reg_j = jnp.where(np_ > 0.0,
                              rs / jnp.maximum(np_ * 4.0, 1.0), 0.0)
            cls_tot = cls_tot + cls_j
            det_tot = det_tot + reg_j
        cls_out[0] = cls_tot / 8.0
        det_out[0] = det_tot / 8.0 * 50.0


def kernel(detection_boxes, detection_labels, regression, classification,
           anchors):
    B, A, C = classification.shape
    nb = detection_boxes.shape[1]
    nblk = A // _BLKA
    padn = _AP - A

    labels = detection_labels.astype(jnp.int32).reshape(B, 1, nb)
    at = jnp.transpose(anchors[0], (1, 0))
    padblk = jnp.tile(jnp.array([[0.0], [0.0], [1.0], [1.0]],
                                jnp.float32), (1, padn))
    at_p = jnp.concatenate([at, padblk], axis=1).reshape(4, _G, 128)
    rt = jnp.transpose(regression, (0, 2, 1))
    rt_p = jnp.pad(rt, ((0, 0), (0, 0), (0, padn))).reshape(B, 4, _G, 128)

    posf_p, glab_p, stats = pl.pallas_call(
        _assign_body,
        grid=(B,),
        in_specs=[
            pl.BlockSpec((1, nb, 4), lambda j: (j, 0, 0),
                         memory_space=pltpu.SMEM),
            pl.BlockSpec((1, 1, nb), lambda j: (j, 0, 0),
                         memory_space=pltpu.SMEM),
            pl.BlockSpec((4, _G, 128), lambda j: (0, 0, 0)),
            pl.BlockSpec((1, 4, _G, 128), lambda j: (j, 0, 0, 0)),
        ],
        out_specs=[
            pl.BlockSpec((1, _G, 128), lambda j: (j, 0, 0)),
            pl.BlockSpec((1, _G, 128), lambda j: (j, 0, 0)),
            pl.BlockSpec((1, 1, 3), lambda j: (j, 0, 0),
                         memory_space=pltpu.SMEM),
        ],
        out_shape=[
            jax.ShapeDtypeStruct((B, _G, 128), jnp.float32),
            jax.ShapeDtypeStruct((B, _G, 128), jnp.float32),
            jax.ShapeDtypeStruct((B, 1, 3), jnp.float32),
        ],
    )(detection_boxes, labels, at_p, rt_p)

    posf_r = posf_p.reshape(B, _AP)[:, :A].reshape(B, nblk, 1, _BLKA)
    glab_r = glab_p.reshape(B, _AP)[:, :A].reshape(B, nblk, 1, _BLKA)

    cls_loss, det_loss = pl.pallas_call(
        _cls_body,
        grid=(B, nblk),
        in_specs=[
            pl.BlockSpec((1, _BLKA, C), lambda j, b: (j, b, 0)),
            pl.BlockSpec((1, 1, 1, _BLKA), lambda j, b: (j, b, 0, 0)),
            pl.BlockSpec((1, 1, 1, _BLKA), lambda j, b: (j, b, 0, 0)),
            pl.BlockSpec(memory_space=pltpu.SMEM),
        ],
        out_specs=[
            pl.BlockSpec(memory_space=pltpu.SMEM),
            pl.BlockSpec(memory_space=pltpu.SMEM),
        ],
        out_shape=[
            jax.ShapeDtypeStruct((1,), jnp.float32),
            jax.ShapeDtypeStruct((1,), jnp.float32),
        ],
        scratch_shapes=[pltpu.SMEM((B, 2), jnp.float32)],
    )(classification, glab_r, posf_r, stats)

    return (cls_loss, det_loss)


# speedup vs baseline: 5.8926x; 1.0050x over previous
import jax
import jax.numpy as jnp
from jax import lax
from jax.experimental import pallas as pl
from jax.experimental.pallas import tpu as pltpu

_BLKA = 20000
_AP = 20480
_G = _AP // 128


def _assign_body(boxes_ref, labs_ref, anch_ref, reg_ref,
                 posf_ref, glab_ref, stats_ref):
    nb = boxes_ref.shape[1]
    ay1 = anch_ref[0]
    ax1 = anch_ref[1]
    ay2 = anch_ref[2]
    ax2 = anch_ref[3]
    aw = ax2 - ax1
    ah = ay2 - ay1
    area_a = ah * aw

    best = jnp.full(ax1.shape, -2.0, jnp.float32)
    gx1 = jnp.zeros(ax1.shape, jnp.float32)
    gy1 = gx1
    gx2 = gx1
    gy2 = gx1
    glabf = gx1
    hk = jnp.int32(0)
    for n in range(nb):
        bx1 = boxes_ref[0, n, 0]
        by1 = boxes_ref[0, n, 1]
        bx2 = boxes_ref[0, n, 2]
        by2 = boxes_ref[0, n, 3]
        labn = labs_ref[0, 0, n]
        validn = labn != 0
        hk = hk | jnp.where(validn, 1, 0)
        iw = jnp.maximum(jnp.minimum(ax2, bx2) - jnp.maximum(ax1, bx1), 0.0)
        ih = jnp.maximum(jnp.minimum(ay2, by2) - jnp.maximum(ay1, by1), 0.0)
        inter = iw * ih
        area_b = (bx2 - bx1) * (by2 - by1)
        ua = jnp.maximum(area_a + area_b - inter, 1e-8)
        iou = inter / ua
        iou = jnp.where(validn, iou, -1.0)
        upd = iou > best
        best = jnp.where(upd, iou, best)
        gx1 = jnp.where(upd, bx1, gx1)
        gy1 = jnp.where(upd, by1, gy1)
        gx2 = jnp.where(upd, bx2, gx2)
        gy2 = jnp.where(upd, by2, gy2)
        glabf = jnp.where(upd, (labn - 1).astype(jnp.float32), glabf)

    big = (gx2 - gx1) * (gy2 - gy1) > 100.0
    pos = (big & (best >= 0.5)) | ((~big) & (best >= 0.15))
    aidx = lax.broadcasted_iota(jnp.int32, ax1.shape, 0) * 128 \
        + lax.broadcasted_iota(jnp.int32, ax1.shape, 1)
    posf = jnp.where(pos & (aidx < 20000), 1.0, 0.0)
    npos = jnp.sum(posf)

    gw_raw = gx2 - gx1
    gh_raw = gy2 - gy1
    gcx = gx1 + 0.5 * gw_raw
    gcy = gy1 + 0.5 * gh_raw
    gw = jnp.maximum(gw_raw, 1.0)
    gh = jnp.maximum(gh_raw, 1.0)
    acx = ax1 + 0.5 * aw
    acy = ay1 + 0.5 * ah
    tdy = (gcy - acy) / ah
    tdx = (gcx - acx) / aw
    tdh = jnp.log(gh / ah)
    tdw = jnp.log(gw / aw)

    def smooth_l1(tcol, k):
        d = jnp.abs(tcol - reg_ref[0, k])
        return jnp.where(d <= 1.0 / 9.0, 4.5 * d * d, d - 0.5 / 9.0)

    rl = smooth_l1(tdy, 0) + smooth_l1(tdx, 1) + smooth_l1(tdh, 2) \
        + smooth_l1(tdw, 3)
    reg_sum = jnp.sum(rl * posf)

    posf_ref[0] = posf
    glab_ref[0] = glabf
    stats_ref[0, 0, 0] = npos
    stats_ref[0, 0, 1] = reg_sum
    stats_ref[0, 0, 2] = jnp.where(hk > 0, 1.0, 0.0)


def _cls_body(cls_ref, glab_ref, pos_ref, stats_ref,
              cls_out, det_out, acc_ref):
    j = pl.program_id(0)
    b = pl.program_id(1)
    nbatch = pl.num_programs(0)
    nblk = pl.num_programs(1)

    p = jnp.clip(cls_ref[0], 0.0001, 1.0 - 0.0001)
    lg = jnp.log(1.0 - p)
    s_blk = -0.75 * jnp.sum(p * p * lg)

    glab_row = glab_ref[0, 0].astype(jnp.int32)
    glab_col = jnp.transpose(glab_row, (1, 0))
    cidx = lax.broadcasted_iota(jnp.int32, p.shape, 1)
    pm = jnp.where(cidx == glab_col, p, 0.0)
    p1_row = lax.dot_general(
        jnp.ones((1, p.shape[1]), jnp.float32), pm,
        (((1,), (1,)), ((), ())),
        precision=lax.Precision.HIGHEST,
        preferred_element_type=jnp.float32)
    posf_row = pos_ref[0, 0]
    p1s = jnp.where(posf_row > 0.0, p1_row, 0.5)
    e1 = 0.75 * p1s * p1s * (-jnp.log(1.0 - p1s))
    f1 = 0.25 * (1.0 - p1s) * (1.0 - p1s) * (-jnp.log(p1s))
    corr_blk = jnp.sum((e1 - f1) * posf_row)

    @pl.when(b == 0)
    def _():
        acc_ref[j, 0] = s_blk
        acc_ref[j, 1] = corr_blk

    @pl.when(b != 0)
    def _():
        acc_ref[j, 0] = acc_ref[j, 0] + s_blk
        acc_ref[j, 1] = acc_ref[j, 1] + corr_blk

    @pl.when((j == nbatch - 1) & (b == nblk - 1))
    def _():
        nb_total = stats_ref.shape[0]
        cls_tot = jnp.float32(0.0)
        det_tot = jnp.float32(0.0)
        for jj in range(nb_total):
            se = acc_ref[jj, 0]
            co = acc_ref[jj, 1]
            np_ = stats_ref[jj, 0, 0]
            rs = stats_ref[jj, 0, 1]
            hk = stats_ref[jj, 0, 2]
            cls_full = (se - co) / jnp.maximum(np_, 1.0)
            cls_j = jnp.where(hk > 0.0, cls_full, se)
            reg_j = jnp.where(np_ > 0.0,
                              rs / jnp.maximum(np_ * 4.0, 1.0), 0.0)
            cls_tot = cls_tot + cls_j
            det_tot = det_tot + reg_j
        cls_out[0] = cls_tot / nb_total
        det_out[0] = det_tot / nb_total * 50.0


def kernel(detection_boxes, detection_labels, regression, classification,
           anchors):
    B, A, C = classification.shape
    nb = detection_boxes.shape[1]
    nblk = A // _BLKA
    padn = _AP - A

    labels = detection_labels.astype(jnp.int32).reshape(B, 1, nb)
    at = jnp.transpose(anchors[0], (1, 0))
    padblk = jnp.tile(jnp.array([[0.0], [0.0], [1.0], [1.0]],
                                jnp.float32), (1, padn))
    at_p = jnp.concatenate([at, padblk], axis=1).reshape(4, _G, 128)
    rt = jnp.transpose(regression, (0, 2, 1))
    rt_p = jnp.pad(rt, ((0, 0), (0, 0), (0, padn))).reshape(B, 4, _G, 128)

    posf_p, glab_p, stats = pl.pallas_call(
        _assign_body,
        grid=(B,),
        in_specs=[
            pl.BlockSpec((1, nb, 4), lambda j: (j, 0, 0),
                         memory_space=pltpu.SMEM),
            pl.BlockSpec((1, 1, nb), lambda j: (j, 0, 0),
                         memory_space=pltpu.SMEM),
            pl.BlockSpec((4, _G, 128), lambda j: (0, 0, 0)),
            pl.BlockSpec((1, 4, _G, 128), lambda j: (j, 0, 0, 0)),
        ],
        out_specs=[
            pl.BlockSpec((1, _G, 128), lambda j: (j, 0, 0)),
            pl.BlockSpec((1, _G, 128), lambda j: (j, 0, 0)),
            pl.BlockSpec((1, 1, 3), lambda j: (j, 0, 0),
                         memory_space=pltpu.SMEM),
        ],
        out_shape=[
            jax.ShapeDtypeStruct((B, _G, 128), jnp.float32),
            jax.ShapeDtypeStruct((B, _G, 128), jnp.float32),
            jax.ShapeDtypeStruct((B, 1, 3), jnp.float32),
        ],
    )(detection_boxes, labels, at_p, rt_p)

    posf_r = posf_p.reshape(B, _AP)[:, :A].reshape(B, nblk, 1, _BLKA)
    glab_r = glab_p.reshape(B, _AP)[:, :A].reshape(B, nblk, 1, _BLKA)

    cls_loss, det_loss = pl.pallas_call(
        _cls_body,
        grid=(B, nblk),
        in_specs=[
            pl.BlockSpec((1, _BLKA, C), lambda j, b: (j, b, 0)),
            pl.BlockSpec((1, 1, 1, _BLKA), lambda j, b: (j, b, 0, 0)),
            pl.BlockSpec((1, 1, 1, _BLKA), lambda j, b: (j, b, 0, 0)),
            pl.BlockSpec(memory_space=pltpu.SMEM),
        ],
        out_specs=[
            pl.BlockSpec(memory_space=pltpu.SMEM),
            pl.BlockSpec(memory_space=pltpu.SMEM),
        ],
        out_shape=[
            jax.ShapeDtypeStruct((1,), jnp.float32),
            jax.ShapeDtypeStruct((1,), jnp.float32),
        ],
        scratch_shapes=[pltpu.SMEM((B, 2), jnp.float32)],
    )(classification, glab_r, posf_r, stats)

    return (cls_loss, det_loss)
